# baseline (speedup 1.0000x reference)
import functools

import jax
import jax.numpy as jnp
import numpy as _np
from jax import lax
from jax.experimental import pallas as pl
from jax.experimental.pallas import tpu as pltpu
from jax.experimental.pallas import tpu_sc as plsc

N = 10000
H = 500
Hp = 512
R = 100
NB = 100
BI = 5
BO = 5
E = 160000
NREL = R + 1
CHUNKS = Hp // 128
CC = 128
ZROWS = N * NREL * CHUNKS

NC = 2
NS = 16
L = 16
EC = 128
NCH = -(-E // (NS * EC))
E_PAD = NS * NCH * EC
NT = NS * NCH
RPT = N // NS
FR = 25


_TM = 2000


def _mm_body(x_ref, w_ref, *o_refs):
    res = jnp.dot(x_ref[...], w_ref[0],
                  preferred_element_type=jnp.float32).astype(jnp.bfloat16)
    for c, o_ref in enumerate(o_refs):
        o_ref[...] = res[:, c * CC:(c + 1) * CC]


def _transform_all(x, wd):
    spec = pl.BlockSpec((_TM, CC), lambda m, r: (r * (N // _TM) + m, 0))
    return pl.pallas_call(
        _mm_body,
        grid=(N // _TM, NREL),
        in_specs=[
            pl.BlockSpec((_TM, Hp), lambda m, r: (m, 0)),
            pl.BlockSpec((1, Hp, Hp), lambda m, r: (r, 0, 0)),
        ],
        out_specs=[spec] * CHUNKS,
        out_shape=[jax.ShapeDtypeStruct((NREL * N, CC), jnp.bfloat16)]
        * CHUNKS,
    )(x, wd)


def _interleave_cols(a):
    s = a.shape[:-1]
    return a.reshape(*s, Hp // 32, 2, 16).swapaxes(-2, -1).reshape(*s, Hp)


def _deinterleave_cols(a):
    s = a.shape[:-1]
    return a.reshape(*s, Hp // 32, 16, 2).swapaxes(-2, -1).reshape(*s, Hp)


def _make_sc_kernel(relu):
    mesh = plsc.VectorSubcoreMesh(core_axis_name="c", subcore_axis_name="s",
                                  num_cores=NC, num_subcores=NS)

    @functools.partial(
        pl.kernel,
        out_type=jax.ShapeDtypeStruct((N, Hp), jnp.float32),
        mesh=mesh,
        compiler_params=pltpu.CompilerParams(use_tc_tiling_on_sc=False,
                                             needs_layout_passes=False),
        scratch_types=[
            pltpu.VMEM((EC,), jnp.int32),
            pltpu.VMEM((EC,), jnp.int32),
            pltpu.VMEM((EC,), jnp.float32),
            pltpu.VMEM((EC, CC), jnp.bfloat16),
            pltpu.VMEM((EC, CC), jnp.float32),
            pltpu.VMEM((FR, CC), jnp.float32),
            pltpu.VMEM((FR, CC), jnp.float32),
            pltpu.VMEM_SHARED((N, CC), jnp.float32),
            pltpu.SemaphoreType.DMA,
        ],
    )
    def sc_kernel(z0, z1, z2, z3, idx2, dst2, norm2, init, out,
                  idx_v, dst_v, norm_v, rows_b, rows_f, fa, fb, acc, sem):
        core = lax.axis_index("c")
        sub = lax.axis_index("s")

        def accumulate(tbl):
            def chunk_body(j, _):
                row = sub * NCH + j
                pltpu.sync_copy(idx2.at[row], idx_v)
                pltpu.sync_copy(dst2.at[row], dst_v)
                pltpu.sync_copy(norm2.at[row], norm_v)
                pltpu.async_copy(tbl.at[idx_v], rows_b, sem).wait()

                def edge_body(e, _):
                    nv = plsc.load_gather(
                        norm_v, [jnp.full((L,), e, jnp.int32)])
                    for g in range(CC // (2 * L)):
                        v = plsc.bitcast(rows_b[e, pl.ds(2 * L * g, 2 * L)],
                                         jnp.int32)
                        lo = plsc.bitcast(jnp.left_shift(v, 16), jnp.float32)
                        hi = plsc.bitcast(
                            jnp.bitwise_and(v, jnp.int32(-65536)), jnp.float32)
                        rows_f[e, pl.ds(2 * L * g, L)] = lo * nv
                        rows_f[e, pl.ds(2 * L * g + L, L)] = hi * nv
                    return 0
                lax.fori_loop(0, EC, edge_body, 0)

                pltpu.sync_copy(rows_f, acc.at[dst_v], add=True)
                return 0
            lax.fori_loop(0, NCH, chunk_body, 0)

        for p in range(CHUNKS // NC):
            cc_idx = core * (CHUNKS // NC) + p

            def zero_row(i, _):
                for kk in range(CC // L):
                    fa[i, pl.ds(kk * L, L)] = jnp.zeros((L,), jnp.float32)
                return 0
            lax.fori_loop(0, FR, zero_row, 0)
            for f in range(RPT // FR):
                pltpu.sync_copy(fa, acc.at[pl.ds(sub * RPT + f * FR, FR)])
            plsc.subcore_barrier()

            tbl_c0 = (z0, z1)[p]
            tbl_c1 = (z2, z3)[p]
            pl.when(core == 0)(lambda: accumulate(tbl_c0))
            pl.when(core == 1)(lambda: accumulate(tbl_c1))
            plsc.subcore_barrier()

            for f in range(RPT // FR):
                r0 = sub * RPT + f * FR
                pltpu.sync_copy(acc.at[pl.ds(r0, FR)], fa)
                pltpu.sync_copy(init.at[pl.ds(r0, FR), pl.ds(cc_idx * CC, CC)],
                                fb)

                def add_row(i, _):
                    for kk in range(CC // L):
                        sl = pl.ds(kk * L, L)
                        v = fa[i, sl] + fb[i, sl]
                        if relu:
                            v = jnp.maximum(v, 0.0)
                        fa[i, sl] = v
                    return 0
                lax.fori_loop(0, FR, add_row, 0)

                pltpu.sync_copy(fa, out.at[pl.ds(r0, FR),
                                           pl.ds(cc_idx * CC, CC)])
            plsc.subcore_barrier()

    return sc_kernel


_make_sc_kernel = functools.lru_cache(maxsize=None)(_make_sc_kernel)


def _expand_weights(w, loop_w):
    eye = jnp.eye(NB, dtype=w.dtype)
    dense = (w[:, :, :, None, :] * eye[None, :, None, :, None])
    dense = dense.reshape(R, H, H)
    dense = jnp.pad(dense, ((0, 0), (0, Hp - H), (0, Hp - H)))
    loop_p = jnp.pad(loop_w, ((0, Hp - H), (0, Hp - H)))
    wd = jnp.concatenate([dense, loop_p[None]], axis=0)
    return _interleave_cols(wd).astype(jnp.bfloat16)


def kernel(nids, edge_index, etypes, norm, emb, w1, loop1, b1, w2, loop2, b2):
    x = jnp.take(emb, nids, axis=0)
    x = jnp.pad(x, ((0, 0), (0, Hp - H)))

    src = edge_index[0]
    dst = edge_index[1]

    pad = E_PAD - E
    base = etypes * N + src
    idx2 = jnp.concatenate([base, jnp.zeros((pad,), jnp.int32)]).reshape(
        NT, EC)
    dst2 = jnp.concatenate([dst, jnp.zeros((pad,), jnp.int32)]).reshape(NT, EC)
    norm2 = jnp.concatenate([norm[:, 0],
                             jnp.zeros((pad,), jnp.float32)]).reshape(NT, EC)

    wd1 = _expand_weights(w1, loop1)
    wd2 = _expand_weights(w2, loop2)
    b1p = jnp.pad(b1, (0, Hp - H))
    b2p = jnp.pad(b2, (0, Hp - H))

    def make_init(zc, bp):
        sloop = jnp.concatenate([t[R * N:] for t in zc], axis=1)
        return _deinterleave_cols(sloop).astype(jnp.float32) + bp[None, :]

    zc1 = _transform_all(x.astype(jnp.bfloat16), wd1)
    h1 = _make_sc_kernel(True)(*zc1, idx2, dst2, norm2, make_init(zc1, b1p))

    zc2 = _transform_all(h1.astype(jnp.bfloat16), wd2)
    h2 = _make_sc_kernel(False)(*zc2, idx2, dst2, norm2, make_init(zc2, b2p))

    return h2[:, :H]


# speedup vs baseline: 1.4959x; 1.4959x over previous
import functools

import jax
import jax.numpy as jnp
from jax import lax
from jax.experimental import pallas as pl
from jax.experimental.pallas import tpu as pltpu
from jax.experimental.pallas import tpu_sc as plsc

N = 10000
H = 500
Hp = 512
R = 100
NB = 100
BI = 5
BO = 5
E = 160000
NREL = R + 1
CHUNKS = Hp // 128
CC = 128
ZROWS = N * NREL * CHUNKS

NC = 2
NS = 16
L = 16
EC = 128
NCH = -(-E // (NS * EC))
E_PAD = NS * NCH * EC
NT = NS * NCH
RPT = N // NS
FR = 125


def _mm_body(x_ref, w_ref, o_ref):
    o_ref[...] = jnp.dot(x_ref[...], w_ref[0],
                         preferred_element_type=jnp.float32)


def _transform_all(x, wd):
    TM = 2000
    return pl.pallas_call(
        _mm_body,
        grid=(N // TM, NREL),
        in_specs=[
            pl.BlockSpec((TM, Hp), lambda m, r: (m, 0)),
            pl.BlockSpec((1, Hp, Hp), lambda m, r: (r, 0, 0)),
        ],
        out_specs=pl.BlockSpec((TM, Hp), lambda m, r: (m, r)),
        out_shape=jax.ShapeDtypeStruct((N, NREL * Hp), jnp.float32),
    )(x, wd)


def _make_sc_kernel(relu):
    mesh = plsc.VectorSubcoreMesh(core_axis_name="c", subcore_axis_name="s",
                                  num_cores=NC, num_subcores=NS)

    @functools.partial(
        pl.kernel,
        out_type=jax.ShapeDtypeStruct((N, Hp), jnp.float32),
        mesh=mesh,
        compiler_params=pltpu.CompilerParams(use_tc_tiling_on_sc=False,
                                             needs_layout_passes=False),
        scratch_types=[
            pltpu.VMEM((EC,), jnp.int32),
            pltpu.VMEM((EC,), jnp.int32),
            pltpu.VMEM((EC,), jnp.float32),
            pltpu.VMEM((EC, CC), jnp.float32),
            pltpu.VMEM((FR, CC), jnp.float32),
            pltpu.VMEM((FR, CC), jnp.float32),
            pltpu.VMEM_SHARED((N, CC), jnp.float32),
            pltpu.SemaphoreType.DMA,
        ],
    )
    def sc_kernel(zr, idx4, dst2, norm2, init, out,
                  idx_v, dst_v, norm_v, rows_v, fa, fb, acc, sem):
        core = lax.axis_index("c")
        sub = lax.axis_index("s")

        for p in range(CHUNKS // NC):
            cc_idx = core * (CHUNKS // NC) + p

            def zero_row(i, _):
                for kk in range(CC // L):
                    fa[i, pl.ds(kk * L, L)] = jnp.zeros((L,), jnp.float32)
                return 0
            lax.fori_loop(0, FR, zero_row, 0)
            for f in range(RPT // FR):
                pltpu.sync_copy(fa, acc.at[pl.ds(sub * RPT + f * FR, FR)])
            plsc.subcore_barrier()

            def chunk_body(j, _):
                row = sub * NCH + j
                pltpu.sync_copy(idx4.at[cc_idx, row], idx_v)
                pltpu.sync_copy(dst2.at[row], dst_v)
                pltpu.sync_copy(norm2.at[row], norm_v)
                pltpu.async_copy(zr.at[idx_v], rows_v, sem).wait()

                def edge_body(e, _):
                    nv = plsc.load_gather(
                        norm_v, [jnp.full((L,), e, jnp.int32)])
                    for kk in range(CC // L):
                        sl = pl.ds(kk * L, L)
                        rows_v[e, sl] = rows_v[e, sl] * nv
                    return 0
                lax.fori_loop(0, EC, edge_body, 0)

                pltpu.sync_copy(rows_v, acc.at[dst_v], add=True)
                return 0
            lax.fori_loop(0, NCH, chunk_body, 0)
            plsc.subcore_barrier()

            for f in range(RPT // FR):
                r0 = sub * RPT + f * FR
                pltpu.sync_copy(acc.at[pl.ds(r0, FR)], fa)
                pltpu.sync_copy(init.at[pl.ds(r0, FR), pl.ds(cc_idx * CC, CC)],
                                fb)

                def add_row(i, _):
                    for kk in range(CC // L):
                        sl = pl.ds(kk * L, L)
                        v = fa[i, sl] + fb[i, sl]
                        if relu:
                            v = jnp.maximum(v, 0.0)
                        fa[i, sl] = v
                    return 0
                lax.fori_loop(0, FR, add_row, 0)

                pltpu.sync_copy(fa, out.at[pl.ds(r0, FR),
                                           pl.ds(cc_idx * CC, CC)])
            plsc.subcore_barrier()

    return sc_kernel


_make_sc_kernel = functools.lru_cache(maxsize=None)(_make_sc_kernel)


def _expand_weights(w, loop_w):
    eye = jnp.eye(NB, dtype=w.dtype)
    dense = (w[:, :, :, None, :] * eye[None, :, None, :, None])
    dense = dense.reshape(R, H, H)
    dense = jnp.pad(dense, ((0, 0), (0, Hp - H), (0, Hp - H)))
    loop_p = jnp.pad(loop_w, ((0, Hp - H), (0, Hp - H)))
    return jnp.concatenate([dense, loop_p[None]], axis=0).astype(jnp.bfloat16)


def kernel(nids, edge_index, etypes, norm, emb, w1, loop1, b1, w2, loop2, b2):
    x = jnp.take(emb, nids, axis=0)
    x = jnp.pad(x, ((0, 0), (0, Hp - H)))

    src = edge_index[0]
    dst = edge_index[1]

    pad = E_PAD - E
    base = (src * NREL + etypes) * CHUNKS
    base = jnp.concatenate([base, jnp.zeros((pad,), jnp.int32)])
    idx4 = (base[None, :] +
            jnp.arange(CHUNKS, dtype=jnp.int32)[:, None]).reshape(
                CHUNKS, NT, EC)
    dst2 = jnp.concatenate([dst, jnp.zeros((pad,), jnp.int32)]).reshape(NT, EC)
    norm2 = jnp.concatenate([norm[:, 0],
                             jnp.zeros((pad,), jnp.float32)]).reshape(NT, EC)

    wd1 = _expand_weights(w1, loop1)
    wd2 = _expand_weights(w2, loop2)
    b1p = jnp.pad(b1, (0, Hp - H))
    b2p = jnp.pad(b2, (0, Hp - H))

    z1 = _transform_all(x.astype(jnp.bfloat16), wd1)
    init1 = z1[:, R * Hp:] + b1p[None, :]
    h1 = _make_sc_kernel(True)(z1.reshape(ZROWS, CC), idx4, dst2, norm2,
                               init1)

    z2 = _transform_all(h1.astype(jnp.bfloat16), wd2)
    init2 = z2[:, R * Hp:] + b2p[None, :]
    h2 = _make_sc_kernel(False)(z2.reshape(ZROWS, CC), idx4, dst2, norm2,
                                init2)

    return h2[:, :H]
